# Initial kernel scaffold; baseline (speedup 1.0000x reference)
import jax
import jax.numpy as jnp
from jax.experimental import pallas as pl


def kernel(z, edge):
    raise NotImplementedError("write your pallas kernel here")


# speedup vs baseline: 2.8703x; 2.8703x over previous
import jax
import jax.numpy as jnp
from jax import lax
from jax.experimental import pallas as pl
from jax.experimental.pallas import tpu as pltpu, tpu_sc as plsc

_NC = 2
_NS = 16
_NW = _NC * _NS
_D = 128
_L = 16
_C = 80


def _edge_dot_kernel(n_edges):
    per_w = n_edges // _NW
    n_chunks = per_w // _C
    assert per_w % _C == 0

    mesh = plsc.VectorSubcoreMesh(core_axis_name="c", subcore_axis_name="s")

    @jax.jit
    def run(z, src, dst):
        @pl.kernel(
            out_type=jax.ShapeDtypeStruct((n_edges,), jnp.float32),
            mesh=mesh,
            compiler_params=pltpu.CompilerParams(needs_layout_passes=False),
            scratch_types=[
                pltpu.VMEM((_C,), jnp.int32),
                pltpu.VMEM((_C,), jnp.int32),
                pltpu.VMEM((_C, _D), jnp.float32),
                pltpu.VMEM((_C, _D), jnp.float32),
                pltpu.VMEM((_C,), jnp.float32),
                pltpu.VMEM((_L * _L,), jnp.float32),
                pltpu.SemaphoreType.DMA,
            ],
        )
        def k(z_hbm, src_hbm, dst_hbm, out_hbm,
              sidx, didx, srows, drows, outv, tbuf, sem):
            wid = lax.axis_index("s") * _NC + lax.axis_index("c")
            base = wid * per_w

            def chunk(i, _):
                cbase = base + i * _C
                pltpu.sync_copy(src_hbm.at[pl.ds(cbase, _C)], sidx)
                pltpu.sync_copy(dst_hbm.at[pl.ds(cbase, _C)], didx)
                cp_s = pltpu.async_copy(z_hbm.at[sidx], srows, sem)
                cp_d = pltpu.async_copy(z_hbm.at[didx], drows, sem)
                cp_s.wait()
                cp_d.wait()

                def group(g, _):
                    eb = g * _L

                    def edge(j, _):
                        e = eb + j
                        acc = srows[e, pl.ds(0, _L)] * drows[e, pl.ds(0, _L)]
                        for t in range(1, _D // _L):
                            acc += (srows[e, pl.ds(t * _L, _L)]
                                    * drows[e, pl.ds(t * _L, _L)])
                        tbuf[pl.ds(j * _L, _L)] = acc
                        return 0

                    lax.fori_loop(0, _L, edge, 0, unroll=2)
                    colidx = lax.iota(jnp.int32, _L) * _L
                    out16 = plsc.load_gather(tbuf, [colidx])
                    for l in range(1, _L):
                        out16 += plsc.load_gather(tbuf, [colidx + l])
                    outv[pl.ds(eb, _L)] = out16
                    return 0

                lax.fori_loop(0, _C // _L, group, 0)
                pltpu.sync_copy(outv, out_hbm.at[pl.ds(cbase, _C)])
                return 0

            lax.fori_loop(0, n_chunks, chunk, 0)

        return k(z, src, dst)

    return run


def kernel(z, edge):
    src = edge[:, 0].astype(jnp.int32)
    dst = edge[:, 1].astype(jnp.int32)
    return _edge_dot_kernel(edge.shape[0])(z, src, dst)


# speedup vs baseline: 6.6190x; 2.3060x over previous
import jax
import jax.numpy as jnp
from jax import lax
from jax.experimental import pallas as pl
from jax.experimental.pallas import tpu as pltpu, tpu_sc as plsc

_NC = 2
_NS = 16
_NW = _NC * _NS
_D = 128
_L = 16
_C = 80


def _edge_dot_kernel(n_edges):
    per_w = n_edges // _NW
    n_chunks = per_w // _C
    assert per_w % _C == 0 and n_chunks % 2 == 1

    mesh = plsc.VectorSubcoreMesh(core_axis_name="c", subcore_axis_name="s")

    @jax.jit
    def run(z, src2, dst2):
        @pl.kernel(
            out_type=jax.ShapeDtypeStruct((n_edges,), jnp.float32),
            mesh=mesh,
            compiler_params=pltpu.CompilerParams(needs_layout_passes=False),
            scratch_types=[
                pltpu.VMEM((n_chunks, _C), jnp.int32),
                pltpu.VMEM((n_chunks, _C), jnp.int32),
                pltpu.VMEM((2, _C, _D), jnp.float32),
                pltpu.VMEM((2, _C, _D), jnp.float32),
                pltpu.VMEM((per_w,), jnp.float32),
                pltpu.VMEM((_L * _L,), jnp.float32),
                pltpu.SemaphoreType.DMA,
                pltpu.SemaphoreType.DMA,
                pltpu.SemaphoreType.DMA,
                pltpu.SemaphoreType.DMA,
            ],
        )
        def k(z_hbm, src_hbm, dst_hbm, out_hbm,
              sidx, didx, srows, drows, outv, tbuf, ss0, ss1, sd0, sd1):
            wid = lax.axis_index("s") * _NC + lax.axis_index("c")
            pltpu.sync_copy(src_hbm.at[wid], sidx)
            pltpu.sync_copy(dst_hbm.at[wid], didx)

            ssems = (ss0, ss1)
            dsems = (sd0, sd1)

            def start(i, b):
                pltpu.async_copy(z_hbm.at[sidx.at[i]], srows.at[b], ssems[b])
                pltpu.async_copy(z_hbm.at[didx.at[i]], drows.at[b], dsems[b])

            def wait(b):
                dummy = z_hbm.at[pl.ds(0, _C)]
                pltpu.make_async_copy(dummy, srows.at[b], ssems[b]).wait()
                pltpu.make_async_copy(dummy, drows.at[b], dsems[b]).wait()

            def compute(g, b):
                sr = srows.at[b]
                dr = drows.at[b]

                def group(gi, _):
                    eb = gi * _L

                    def edge(j, _):
                        e = eb + j
                        acc = sr[e, pl.ds(0, _L)] * dr[e, pl.ds(0, _L)]
                        for t in range(1, _D // _L):
                            acc += (sr[e, pl.ds(t * _L, _L)]
                                    * dr[e, pl.ds(t * _L, _L)])
                        tbuf[pl.ds(j * _L, _L)] = acc
                        return 0

                    lax.fori_loop(0, _L, edge, 0, unroll=2)
                    colidx = lax.iota(jnp.int32, _L) * _L
                    out16 = plsc.load_gather(tbuf, [colidx])
                    for l in range(1, _L):
                        out16 += plsc.load_gather(tbuf, [colidx + l])
                    outv[pl.ds(g * _C + eb, _L)] = out16
                    return 0

                lax.fori_loop(0, _C // _L, group, 0)

            start(0, 0)

            def outer(t, _):
                g0 = t * 2
                for b in range(2):
                    g = g0 + b
                    wait(b)

                    @pl.when(g + 1 < n_chunks)
                    def _():
                        start(g + 1, 1 - b)

                    compute(g, b)
                return 0

            lax.fori_loop(0, (n_chunks - 1) // 2, outer, 0)
            wait(0)
            compute(n_chunks - 1, 0)
            pltpu.sync_copy(outv, out_hbm.at[pl.ds(wid * per_w, per_w)])

        return k(z, src2, dst2)

    return run


def kernel(z, edge):
    n_edges = edge.shape[0]
    per_w = n_edges // _NW
    src2 = edge[:, 0].astype(jnp.int32).reshape(_NW, per_w // _C, _C)
    dst2 = edge[:, 1].astype(jnp.int32).reshape(_NW, per_w // _C, _C)
    return _edge_dot_kernel(n_edges)(z, src2, dst2)


# speedup vs baseline: 6.7233x; 1.0158x over previous
import jax
import jax.numpy as jnp
from jax import lax
from jax.experimental import pallas as pl
from jax.experimental.pallas import tpu as pltpu, tpu_sc as plsc

_NC = 2
_NS = 16
_NW = _NC * _NS
_D = 128
_L = 16
_C = 80


def _edge_dot_kernel(n_edges):
    per_w = n_edges // _NW
    n_chunks = per_w // _C
    assert per_w % _C == 0 and n_chunks % 2 == 1

    mesh = plsc.VectorSubcoreMesh(core_axis_name="c", subcore_axis_name="s")

    @jax.jit
    def run(z, src2, dst2):
        @pl.kernel(
            out_type=jax.ShapeDtypeStruct((n_edges,), jnp.float32),
            mesh=mesh,
            compiler_params=pltpu.CompilerParams(
                needs_layout_passes=False, use_tc_tiling_on_sc=False),
            scratch_types=[
                pltpu.VMEM((n_chunks, _C), jnp.int32),
                pltpu.VMEM((n_chunks, _C), jnp.int32),
                pltpu.VMEM((2, _C, _D // 2), jnp.int32),
                pltpu.VMEM((2, _C, _D // 2), jnp.int32),
                pltpu.VMEM((per_w,), jnp.float32),
                pltpu.VMEM((_L * _L,), jnp.float32),
                pltpu.SemaphoreType.DMA,
                pltpu.SemaphoreType.DMA,
                pltpu.SemaphoreType.DMA,
                pltpu.SemaphoreType.DMA,
            ],
        )
        def k(z_hbm, src_hbm, dst_hbm, out_hbm,
              sidx, didx, srows, drows, outv, tbuf, ss0, ss1, sd0, sd1):
            wid = lax.axis_index("s") * _NC + lax.axis_index("c")
            pltpu.sync_copy(src_hbm.at[wid], sidx)
            pltpu.sync_copy(dst_hbm.at[wid], didx)

            ssems = (ss0, ss1)
            dsems = (sd0, sd1)

            def start(i, b):
                pltpu.async_copy(z_hbm.at[sidx.at[i]], srows.at[b], ssems[b])
                pltpu.async_copy(z_hbm.at[didx.at[i]], drows.at[b], dsems[b])

            def wait(b):
                dummy = z_hbm.at[pl.ds(0, _C)]
                pltpu.make_async_copy(dummy, srows.at[b], ssems[b]).wait()
                pltpu.make_async_copy(dummy, drows.at[b], dsems[b]).wait()

            def compute(g, b):
                sr = srows.at[b]
                dr = drows.at[b]

                def group(gi, _):
                    eb = gi * _L

                    def edge(j, _):
                        e = eb + j
                        acc = None
                        for t in range(_D // (2 * _L)):
                            a = plsc.bitcast(sr[e, pl.ds(t * _L, _L)],
                                             jnp.bfloat16)
                            b = plsc.bitcast(dr[e, pl.ds(t * _L, _L)],
                                             jnp.bfloat16)
                            a0, a1 = plsc.unpack(
                                a, format=plsc.PackFormat.INTERLEAVED,
                                preferred_element_type=jnp.float32)
                            b0, b1 = plsc.unpack(
                                b, format=plsc.PackFormat.INTERLEAVED,
                                preferred_element_type=jnp.float32)
                            p = a0 * b0 + a1 * b1
                            acc = p if acc is None else acc + p
                        tbuf[pl.ds(j * _L, _L)] = acc
                        return 0

                    lax.fori_loop(0, _L, edge, 0, unroll=2)
                    colidx = lax.iota(jnp.int32, _L) * _L
                    out16 = plsc.load_gather(tbuf, [colidx])
                    for l in range(1, _L):
                        out16 += plsc.load_gather(tbuf, [colidx + l])
                    outv[pl.ds(g * _C + eb, _L)] = out16
                    return 0

                lax.fori_loop(0, _C // _L, group, 0)

            start(0, 0)

            def outer(t, _):
                g0 = t * 2
                for b in range(2):
                    g = g0 + b
                    wait(b)

                    @pl.when(g + 1 < n_chunks)
                    def _():
                        start(g + 1, 1 - b)

                    compute(g, b)
                return 0

            lax.fori_loop(0, (n_chunks - 1) // 2, outer, 0)
            wait(0)
            compute(n_chunks - 1, 0)
            pltpu.sync_copy(outv, out_hbm.at[pl.ds(wid * per_w, per_w)])

        return k(z, src2, dst2)

    return run


def kernel(z, edge):
    n_edges = edge.shape[0]
    per_w = n_edges // _NW
    src2 = edge[:, 0].astype(jnp.int32).reshape(_NW, per_w // _C, _C)
    dst2 = edge[:, 1].astype(jnp.int32).reshape(_NW, per_w // _C, _C)
    zi = lax.bitcast_convert_type(
        z.astype(jnp.bfloat16).reshape(z.shape[0], z.shape[1] // 2, 2),
        jnp.int32)
    return _edge_dot_kernel(n_edges)(zi, src2, dst2)
